# baseline (speedup 1.0000x reference)
import jax
import jax.numpy as jnp
from jax import lax
from jax.experimental import pallas as pl
from jax.experimental.pallas import tpu as pltpu
from jax.experimental.pallas import tpu_sc as plsc

VOCAB = 1000000
HIDDEN = 64
HPAD = 128
L = 200
B = 1024
NW = 32
ROWS_PER_W = B // NW
TOK_PER_W = ROWS_PER_W * L
NCHUNK = ROWS_PER_W
LANES = 16
VPR = HIDDEN // LANES
GDEPTH = 2
ODEPTH = 2


def _body(ids_hbm, word_hbm, pos_hbm, out_hbm, idx_all, posblk, *rest):
    gbuf = rest[:GDEPTH]
    obuf = rest[GDEPTH:GDEPTH + ODEPTH]
    semg = rest[GDEPTH + ODEPTH:2 * GDEPTH + ODEPTH]
    sems = rest[2 * GDEPTH + ODEPTH:]

    wid = lax.axis_index("s") * 2 + lax.axis_index("c")
    base = wid * TOK_PER_W

    pltpu.sync_copy(ids_hbm.at[pl.ds(base, TOK_PER_W)], idx_all)
    pltpu.sync_copy(pos_hbm, posblk)

    def start_gather(c, s):
        off = c * L
        pltpu.async_copy(word_hbm.at[idx_all.at[pl.ds(off, 128)]],
                         gbuf[s].at[pl.ds(0, 128)], semg[s])
        pltpu.async_copy(word_hbm.at[idx_all.at[pl.ds(off + 128, 72)]],
                         gbuf[s].at[pl.ds(128, 72)], semg[s])

    def wait_gather(s):
        pltpu.make_async_copy(word_hbm.at[pl.ds(0, L)], gbuf[s], semg[s]).wait()

    def wait_store(s):
        pltpu.make_async_copy(obuf[s], out_hbm.at[pl.ds(0, L)], sems[s]).wait()

    for c in range(GDEPTH - 1):
        start_gather(c, c)

    def group(g, carry):
        for b in range(GDEPTH):
            c = g * GDEPTH + b
            so = b % ODEPTH

            @pl.when(c + GDEPTH - 1 < NCHUNK)
            def _():
                start_gather(c + GDEPTH - 1, (b + GDEPTH - 1) % GDEPTH)

            wait_gather(b)

            @pl.when(c >= ODEPTH)
            def _():
                wait_store(so)

            ob = obuf[so]
            gb = gbuf[b]

            @plsc.parallel_loop(0, L, step=1, unroll=8)
            def _(r):
                for j in range(VPR):
                    sl = pl.ds(j * LANES, LANES)
                    ob[r, sl] = gb[r, sl] + posblk[pl.ds(r * HIDDEN + j * LANES, LANES)]

            pltpu.async_copy(ob, out_hbm.at[pl.ds(base + c * L, L)], sems[so])
        return carry

    lax.fori_loop(0, NCHUNK // GDEPTH, group, 0)
    for s in range(ODEPTH):
        wait_store(s)


@jax.jit
def kernel(input_ids, word_emb, pos_emb):
    ids_flat = input_ids.reshape(-1).astype(jnp.int32)
    word_pad = jnp.broadcast_to(word_emb[:, None, :],
                                (VOCAB, 2, HIDDEN)).reshape(VOCAB, HPAD)
    pos_lin = pos_emb[1:L + 1].reshape(-1)
    mesh = plsc.VectorSubcoreMesh(core_axis_name="c", subcore_axis_name="s")
    out = pl.kernel(
        _body,
        out_type=jax.ShapeDtypeStruct((B * L, HPAD), jnp.float32),
        mesh=mesh,
        scratch_types=(
            [pltpu.VMEM((TOK_PER_W,), jnp.int32),
             pltpu.VMEM((L * HIDDEN,), jnp.float32)]
            + [pltpu.VMEM((L, HPAD), jnp.float32)] * (GDEPTH + ODEPTH)
            + [pltpu.SemaphoreType.DMA] * (GDEPTH + ODEPTH)
        ),
    )(ids_flat, word_pad, pos_lin)
    return out.reshape(B, L, HPAD)[:, :, :HIDDEN]


# speedup vs baseline: 1.2211x; 1.2211x over previous
import jax
import jax.numpy as jnp
from jax import lax
from jax.experimental import pallas as pl
from jax.experimental.pallas import tpu as pltpu
from jax.experimental.pallas import tpu_sc as plsc

VOCAB = 1000000
HIDDEN = 64
HPAD = 128
L = 200
B = 1024
NW = 32
ROWS_PER_W = B // NW
TOK_PER_W = ROWS_PER_W * L
NCHUNK = ROWS_PER_W
LANES = 16
VPR = HIDDEN // LANES
GDEPTH = 2
ODEPTH = 2


def _body(ids_hbm, word_hbm, pos_hbm, out_hbm, idx_all, posblk, *rest):
    gbuf = rest[:GDEPTH]
    obuf = rest[GDEPTH:GDEPTH + ODEPTH]
    semg = rest[GDEPTH + ODEPTH:2 * GDEPTH + ODEPTH]
    sems = rest[2 * GDEPTH + ODEPTH:]

    wid = lax.axis_index("s") * 2 + lax.axis_index("c")
    base = wid * TOK_PER_W

    pltpu.sync_copy(ids_hbm.at[pl.ds(base, TOK_PER_W)], idx_all)
    pltpu.sync_copy(pos_hbm, posblk)

    def start_gather(c, s):
        off = c * L
        pltpu.async_copy(word_hbm.at[idx_all.at[pl.ds(off, 128)]],
                         gbuf[s].at[pl.ds(0, 128)], semg[s])
        pltpu.async_copy(word_hbm.at[idx_all.at[pl.ds(off + 128, 72)]],
                         gbuf[s].at[pl.ds(128, 72)], semg[s])

    def wait_gather(s):
        pltpu.make_async_copy(word_hbm.at[pl.ds(0, L)], gbuf[s], semg[s]).wait()

    def wait_store(s):
        pltpu.make_async_copy(obuf[s], out_hbm.at[pl.ds(0, L)], sems[s]).wait()

    for c in range(GDEPTH - 1):
        start_gather(c, c)

    def group(g, carry):
        for b in range(GDEPTH):
            c = g * GDEPTH + b
            so = b % ODEPTH

            @pl.when(c + GDEPTH - 1 < NCHUNK)
            def _():
                start_gather(c + GDEPTH - 1, (b + GDEPTH - 1) % GDEPTH)

            wait_gather(b)

            @pl.when(c >= ODEPTH)
            def _():
                wait_store(so)

            ob = obuf[so]
            gb = gbuf[b]

            @plsc.parallel_loop(0, L, step=1, unroll=8)
            def _(r):
                for j in range(VPR):
                    sl = pl.ds(j * LANES, LANES)
                    ob[r, sl] = gb[r, sl] + posblk[pl.ds(r * HIDDEN + j * LANES, LANES)]

            pltpu.async_copy(ob, out_hbm.at[pl.ds(base + c * L, L)], sems[so])
        return carry

    lax.fori_loop(0, NCHUNK // GDEPTH, group, 0)
    for s in range(ODEPTH):
        wait_store(s)


@jax.jit
def kernel(input_ids, word_emb, pos_emb):
    ids_flat = input_ids.reshape(-1).astype(jnp.int32)
    word_pad = jnp.pad(word_emb, ((0, 0), (0, HPAD - HIDDEN)))
    pos_lin = pos_emb[1:L + 1].reshape(-1)
    mesh = plsc.VectorSubcoreMesh(core_axis_name="c", subcore_axis_name="s")
    out = pl.kernel(
        _body,
        out_type=jax.ShapeDtypeStruct((B * L, HPAD), jnp.float32),
        mesh=mesh,
        scratch_types=(
            [pltpu.VMEM((TOK_PER_W,), jnp.int32),
             pltpu.VMEM((L * HIDDEN,), jnp.float32)]
            + [pltpu.VMEM((L, HPAD), jnp.float32)] * (GDEPTH + ODEPTH)
            + [pltpu.SemaphoreType.DMA] * (GDEPTH + ODEPTH)
        ),
    )(ids_flat, word_pad, pos_lin)
    return out.reshape(B, L, HPAD)[:, :, :HIDDEN]
